# baseline (speedup 1.0000x reference)
import functools

import jax
import jax.numpy as jnp
from jax import lax
from jax.experimental import pallas as pl
from jax.experimental.pallas import tpu as pltpu
from jax.experimental.pallas import tpu_sc as plsc

N = 10000
E = 320000
D = 128
NC = 2
NS = 16
NW = NC * NS
EPW = E // NW
CH = 125
NCH = EPW // CH
NR = 10240
RPS = NR // NS
ZR = 128
NV = 2
FL = N * NV
FLP = 20480
FPS = FLP // NS
CH2 = 125
NCH2 = EPW * NV // CH2
DEG_PAD = 10240
DPS = DEG_PAD // NS
RB = 1000


def _sc_mesh():
    return plsc.VectorSubcoreMesh(
        core_axis_name="c", subcore_axis_name="s",
        num_cores=NC, num_subcores=NS)


@functools.cache
def _deg_kernel():
    @functools.partial(
        pl.kernel,
        out_type=jax.ShapeDtypeStruct((NC * DEG_PAD,), jnp.float32),
        mesh=_sc_mesh(),
        scratch_types=[
            pltpu.VMEM((NCH, CH), jnp.int32),
            pltpu.VMEM((CH,), jnp.float32),
            pltpu.VMEM_SHARED((DEG_PAD,), jnp.float32),
        ],
    )
    def deg(ei_hbm, zrow_hbm, ones_hbm, out_hbm, idx_d2, ones_v, acc):
        c = lax.axis_index("c")
        s = lax.axis_index("s")
        w = c * NS + s
        pltpu.sync_copy(ei_hbm.at[1, w], idx_d2)
        pltpu.sync_copy(ones_hbm, ones_v)
        pltpu.sync_copy(zrow_hbm, acc.at[pl.ds(s * DPS, DPS)])
        plsc.subcore_barrier()

        @pl.loop(0, NCH)
        def _(i):
            pltpu.sync_copy(ones_v, acc.at[idx_d2.at[i]], add=True)

        plsc.subcore_barrier()
        pltpu.sync_copy(acc.at[pl.ds(s * DPS, DPS)],
                        out_hbm.at[pl.ds(c * DEG_PAD + s * DPS, DPS)])

    return deg


@functools.cache
def _agg_kernel(d):
    @functools.partial(
        pl.kernel,
        out_type=jax.ShapeDtypeStruct((NC, NR, d), jnp.float32),
        mesh=_sc_mesh(),
        scratch_types=[
            pltpu.VMEM((2, CH), jnp.int32),
            pltpu.VMEM((NCH, CH), jnp.int32),
            pltpu.VMEM((2, CH, d), jnp.float32),
            pltpu.VMEM_SHARED((NR, d), jnp.float32),
            pltpu.SemaphoreType.DMA,
            pltpu.SemaphoreType.DMA,
            pltpu.SemaphoreType.DMA,
            pltpu.SemaphoreType.DMA,
        ],
    )
    def agg(h_hbm, ei_hbm, zblk_hbm, out_hbm,
            sidx, idx_d2, rows, acc, gsem0, gsem1, ssem0, ssem1):
        c = lax.axis_index("c")
        s = lax.axis_index("s")
        w = c * NS + s
        gsems = (gsem0, gsem1)
        ssems = (ssem0, ssem1)

        def load_sidx(i, slot, sem):
            pltpu.async_copy(ei_hbm.at[0, w, i], sidx.at[slot], sem)

        def wait_sidx(slot, sem):
            pltpu.make_async_copy(
                ei_hbm.at[0, w, 0], sidx.at[slot], sem).wait()

        def issue_gather(slot, sem):
            pltpu.async_copy(h_hbm.at[sidx.at[slot]], rows.at[slot], sem)

        def wait_gather(slot, sem):
            pltpu.make_async_copy(
                h_hbm.at[sidx.at[slot]], rows.at[slot], sem).wait()

        def scatter(i, slot):
            pltpu.sync_copy(rows.at[slot], acc.at[idx_d2.at[i]], add=True)

        pltpu.sync_copy(ei_hbm.at[1, w], idx_d2)
        pltpu.sync_copy(ei_hbm.at[0, w, 0], sidx.at[0])
        load_sidx(1, 1, ssems[1])
        issue_gather(0, gsems[0])

        @pl.loop(0, RPS // ZR)
        def _(j):
            pltpu.sync_copy(zblk_hbm, acc.at[pl.ds(s * RPS + j * ZR, ZR)])

        plsc.subcore_barrier()

        @pl.loop(0, (NCH - 2) // 2)
        def _(j):
            for b in range(2):
                i = j * 2 + b
                wait_gather(b, gsems[b])
                wait_sidx(1 - b, ssems[1 - b])
                issue_gather(1 - b, gsems[1 - b])
                scatter(i, b)
                load_sidx(i + 2, b, ssems[b])

        i = NCH - 2
        b = i % 2
        wait_gather(b, gsems[b])
        wait_sidx(1 - b, ssems[1 - b])
        issue_gather(1 - b, gsems[1 - b])
        scatter(i, b)
        wait_gather(1 - b, gsems[1 - b])
        scatter(NCH - 1, 1 - b)

        plsc.subcore_barrier()

        @pl.loop(0, RPS // ZR)
        def _(j):
            r0 = s * RPS + j * ZR
            pltpu.sync_copy(acc.at[pl.ds(r0, ZR)], out_hbm.at[c, pl.ds(r0, ZR)])

    return agg


@functools.cache
def _aggn_kernel():
    @functools.partial(
        pl.kernel,
        out_type=jax.ShapeDtypeStruct((NC * FLP,), jnp.float32),
        mesh=_sc_mesh(),
        scratch_types=[
            pltpu.VMEM((NCH2, CH2), jnp.int32),
            pltpu.VMEM((NCH2, CH2), jnp.int32),
            pltpu.VMEM((2, CH2), jnp.float32),
            pltpu.VMEM_SHARED((FLP,), jnp.float32),
            pltpu.VMEM_SHARED((FLP,), jnp.float32),
            pltpu.SemaphoreType.DMA,
            pltpu.SemaphoreType.DMA,
        ],
    )
    def aggn(t_hbm, gi_hbm, si_hbm, zrow_hbm, out_hbm,
             gidx, sidx, vals, tsp, acc, sem0, sem1):
        c = lax.axis_index("c")
        s = lax.axis_index("s")
        w = c * NS + s
        sems = (sem0, sem1)

        pltpu.sync_copy(gi_hbm.at[w], gidx)
        pltpu.sync_copy(si_hbm.at[w], sidx)
        pltpu.sync_copy(t_hbm.at[pl.ds(s * FPS, FPS)],
                        tsp.at[pl.ds(s * FPS, FPS)])
        pltpu.sync_copy(zrow_hbm, acc.at[pl.ds(s * FPS, FPS)])
        plsc.subcore_barrier()

        def issue_gather(i, slot):
            pltpu.async_copy(tsp.at[gidx.at[i]], vals.at[slot], sems[slot])

        def wait_gather(slot):
            pltpu.make_async_copy(
                tsp.at[gidx.at[0]], vals.at[slot], sems[slot]).wait()

        def scatter(i, slot):
            pltpu.sync_copy(vals.at[slot], acc.at[sidx.at[i]], add=True)

        issue_gather(0, 0)
        issue_gather(1, 1)

        @pl.loop(0, (NCH2 - 2) // 2)
        def _(j):
            for b in range(2):
                i = j * 2 + b
                wait_gather(b)
                scatter(i, b)
                issue_gather(i + 2, b)

        wait_gather(0)
        scatter(NCH2 - 2, 0)
        wait_gather(1)
        scatter(NCH2 - 1, 1)

        plsc.subcore_barrier()
        pltpu.sync_copy(acc.at[pl.ds(s * FPS, FPS)],
                        out_hbm.at[pl.ds(c * FLP + s * FPS, FPS)])

    return aggn


def _dinv_block(degp_ref):
    deg = degp_ref[0] + degp_ref[1] + 1.0
    return lax.rsqrt(deg)


def _mm_scale_body(x_ref, w_ref, degp_ref, o_ref):
    h = jnp.dot(x_ref[...], w_ref[...], preferred_element_type=jnp.float32)
    o_ref[...] = h * _dinv_block(degp_ref)


def _epilogue1_body(aggp_ref, h1p_ref, degp_ref, b1_ref, w2_ref, o_ref):
    dinv = _dinv_block(degp_ref)
    s = aggp_ref[0] + aggp_ref[1] + h1p_ref[...]
    h = jnp.maximum(s * dinv + b1_ref[...], 0.0)
    o_ref[...] = jnp.dot(h * dinv, w2_ref[...],
                         preferred_element_type=jnp.float32)


def _epilogue2_body(aggp_ref, tp_ref, degp_ref, b2_ref, o_ref):
    dinv = _dinv_block(degp_ref)
    s = aggp_ref[0] + aggp_ref[1] + tp_ref[...]
    o_ref[...] = s * dinv + b2_ref[...]


_GRID = N // RB
_spec_rows = pl.BlockSpec((RB, D), lambda i: (i, 0))
_spec_nrow = pl.BlockSpec((RB, NV), lambda i: (i, 0))
_spec_pair = pl.BlockSpec((NC, RB, D), lambda i: (0, i, 0))
_spec_npair = pl.BlockSpec((NC, RB, NV), lambda i: (0, i, 0))
_spec_deg = pl.BlockSpec((NC, RB, 1), lambda i: (0, i, 0))
_spec_w = pl.BlockSpec((D, D), lambda i: (0, 0))
_spec_wn = pl.BlockSpec((D, NV), lambda i: (0, 0))
_spec_b = pl.BlockSpec((1, D), lambda i: (0, 0))
_spec_bn = pl.BlockSpec((1, NV), lambda i: (0, 0))
_out_rows = jax.ShapeDtypeStruct((N, D), jnp.float32)
_out_nrows = jax.ShapeDtypeStruct((N, NV), jnp.float32)


def kernel(x, edge_index, W1, b1, W2, b2):
    ei = edge_index.reshape(2, NW, NCH, CH)
    gflat = jnp.stack([2 * edge_index[0], 2 * edge_index[0] + 1],
                      axis=-1).reshape(NW, NCH2, CH2)
    sflat = jnp.stack([2 * edge_index[1], 2 * edge_index[1] + 1],
                      axis=-1).reshape(NW, NCH2, CH2)
    zrow = jnp.zeros((DPS,), jnp.float32)
    zfps = jnp.zeros((FPS,), jnp.float32)
    ones = jnp.ones((CH,), jnp.float32)
    zblk = jnp.zeros((ZR, D), jnp.float32)
    b1r = b1.reshape(1, D)
    b2r = b2.reshape(1, NV)

    degp = _deg_kernel()(ei, zrow, ones)
    degp3 = degp.reshape(NC, DEG_PAD, 1)

    h1p = pl.pallas_call(
        _mm_scale_body,
        grid=(_GRID,),
        in_specs=[_spec_rows, _spec_w, _spec_deg],
        out_specs=_spec_rows,
        out_shape=_out_rows,
    )(x, W1, degp3)

    aggp1 = _agg_kernel(D)(h1p, ei, zblk)

    tp = pl.pallas_call(
        _epilogue1_body,
        grid=(_GRID,),
        in_specs=[_spec_pair, _spec_rows, _spec_deg, _spec_b, _spec_wn],
        out_specs=_spec_nrow,
        out_shape=_out_nrows,
    )(aggp1, h1p, degp3, b1r, W2)

    tflat = jnp.pad(tp.reshape(FL), (0, FLP - FL))
    aggp2 = _aggn_kernel()(tflat, gflat, sflat, zfps)
    aggp2 = aggp2.reshape(NC, FLP // NV, NV)

    outp = pl.pallas_call(
        _epilogue2_body,
        grid=(_GRID,),
        in_specs=[_spec_npair, _spec_nrow, _spec_deg, _spec_bn],
        out_specs=_spec_nrow,
        out_shape=_out_nrows,
    )(aggp2, tp, degp3, b2r)

    return outp


# speedup vs baseline: 1.8827x; 1.8827x over previous
import functools

import jax
import jax.numpy as jnp
from jax import lax
from jax.experimental import pallas as pl
from jax.experimental.pallas import tpu as pltpu
from jax.experimental.pallas import tpu_sc as plsc

N = 10000
E = 320000
D = 128
NC = 2
NS = 16
NW = NC * NS
EPW = E // NW
CH = 125
NCH = EPW // CH
NR = 10240
RPS = NR // NS
ZR = 128
NV = 2
DEG_PAD = 10240
DPS = DEG_PAD // NS
RB = 1000


def _sc_mesh():
    return plsc.VectorSubcoreMesh(
        core_axis_name="c", subcore_axis_name="s",
        num_cores=NC, num_subcores=NS)


@functools.cache
def _deg_kernel():
    @functools.partial(
        pl.kernel,
        out_type=jax.ShapeDtypeStruct((NC * DEG_PAD,), jnp.float32),
        mesh=_sc_mesh(),
        scratch_types=[
            pltpu.VMEM((NCH, CH), jnp.int32),
            pltpu.VMEM((CH,), jnp.float32),
            pltpu.VMEM_SHARED((DEG_PAD,), jnp.float32),
        ],
    )
    def deg(ei_hbm, zrow_hbm, ones_hbm, out_hbm, idx_d2, ones_v, acc):
        c = lax.axis_index("c")
        s = lax.axis_index("s")
        w = c * NS + s
        pltpu.sync_copy(ei_hbm.at[1, w], idx_d2)
        pltpu.sync_copy(ones_hbm, ones_v)
        pltpu.sync_copy(zrow_hbm, acc.at[pl.ds(s * DPS, DPS)])
        plsc.subcore_barrier()

        @pl.loop(0, NCH)
        def _(i):
            pltpu.sync_copy(ones_v, acc.at[idx_d2.at[i]], add=True)

        plsc.subcore_barrier()
        pltpu.sync_copy(acc.at[pl.ds(s * DPS, DPS)],
                        out_hbm.at[pl.ds(c * DEG_PAD + s * DPS, DPS)])

    return deg


@functools.cache
def _agg_kernel(d):
    @functools.partial(
        pl.kernel,
        out_type=jax.ShapeDtypeStruct((NC, NR, d), jnp.float32),
        mesh=_sc_mesh(),
        scratch_types=[
            pltpu.VMEM((2, CH), jnp.int32),
            pltpu.VMEM((NCH, CH), jnp.int32),
            pltpu.VMEM((2, CH, d), jnp.float32),
            pltpu.VMEM_SHARED((NR, d), jnp.float32),
            pltpu.SemaphoreType.DMA,
            pltpu.SemaphoreType.DMA,
            pltpu.SemaphoreType.DMA,
            pltpu.SemaphoreType.DMA,
        ],
    )
    def agg(h_hbm, ei_hbm, zblk_hbm, out_hbm,
            sidx, idx_d2, rows, acc, gsem0, gsem1, ssem0, ssem1):
        c = lax.axis_index("c")
        s = lax.axis_index("s")
        w = c * NS + s
        gsems = (gsem0, gsem1)
        ssems = (ssem0, ssem1)

        def load_sidx(i, slot, sem):
            pltpu.async_copy(ei_hbm.at[0, w, i], sidx.at[slot], sem)

        def wait_sidx(slot, sem):
            pltpu.make_async_copy(
                ei_hbm.at[0, w, 0], sidx.at[slot], sem).wait()

        def issue_gather(slot, sem):
            pltpu.async_copy(h_hbm.at[sidx.at[slot]], rows.at[slot], sem)

        def wait_gather(slot, sem):
            pltpu.make_async_copy(
                h_hbm.at[sidx.at[slot]], rows.at[slot], sem).wait()

        def scatter(i, slot):
            pltpu.sync_copy(rows.at[slot], acc.at[idx_d2.at[i]], add=True)

        pltpu.sync_copy(ei_hbm.at[1, w], idx_d2)
        pltpu.sync_copy(ei_hbm.at[0, w, 0], sidx.at[0])
        load_sidx(1, 1, ssems[1])
        issue_gather(0, gsems[0])

        @pl.loop(0, RPS // ZR)
        def _(j):
            pltpu.sync_copy(zblk_hbm, acc.at[pl.ds(s * RPS + j * ZR, ZR)])

        plsc.subcore_barrier()

        @pl.loop(0, (NCH - 2) // 2)
        def _(j):
            for b in range(2):
                i = j * 2 + b
                wait_gather(b, gsems[b])
                wait_sidx(1 - b, ssems[1 - b])
                issue_gather(1 - b, gsems[1 - b])
                scatter(i, b)
                load_sidx(i + 2, b, ssems[b])

        i = NCH - 2
        b = i % 2
        wait_gather(b, gsems[b])
        wait_sidx(1 - b, ssems[1 - b])
        issue_gather(1 - b, gsems[1 - b])
        scatter(i, b)
        wait_gather(1 - b, gsems[1 - b])
        scatter(NCH - 1, 1 - b)

        plsc.subcore_barrier()

        @pl.loop(0, RPS // ZR)
        def _(j):
            r0 = s * RPS + j * ZR
            pltpu.sync_copy(acc.at[pl.ds(r0, ZR)], out_hbm.at[c, pl.ds(r0, ZR)])

    return agg


@functools.cache
def _aggn_kernel():
    @functools.partial(
        pl.kernel,
        out_type=jax.ShapeDtypeStruct((NC * NV * DEG_PAD,), jnp.float32),
        mesh=_sc_mesh(),
        scratch_types=[
            pltpu.VMEM((NCH, CH), jnp.int32),
            pltpu.VMEM((NCH, CH), jnp.int32),
            pltpu.VMEM((2, CH), jnp.float32),
            pltpu.VMEM_SHARED((DEG_PAD,), jnp.float32),
            pltpu.VMEM_SHARED((DEG_PAD,), jnp.float32),
            pltpu.VMEM_SHARED((DEG_PAD,), jnp.float32),
            pltpu.VMEM_SHARED((DEG_PAD,), jnp.float32),
            pltpu.SemaphoreType.DMA,
            pltpu.SemaphoreType.DMA,
        ],
    )
    def aggn(t_hbm, ei_hbm, zrow_hbm, out_hbm,
             sidx, didx, vals, tsp0, tsp1, acc0, acc1, sem0, sem1):
        c = lax.axis_index("c")
        s = lax.axis_index("s")
        w = c * NS + s

        pltpu.sync_copy(ei_hbm.at[0, w], sidx)
        pltpu.sync_copy(ei_hbm.at[1, w], didx)
        sl = pl.ds(s * DPS, DPS)
        pltpu.sync_copy(t_hbm.at[pl.ds(s * DPS, DPS)], tsp0.at[sl])
        pltpu.sync_copy(t_hbm.at[pl.ds(DEG_PAD + s * DPS, DPS)], tsp1.at[sl])
        pltpu.sync_copy(zrow_hbm, acc0.at[sl])
        pltpu.sync_copy(zrow_hbm, acc1.at[sl])
        plsc.subcore_barrier()

        def run_class(tspc, accc):
            def issue_gather(i, slot, sem):
                pltpu.async_copy(tspc.at[sidx.at[i]], vals.at[slot], sem)

            def wait_gather(slot, sem):
                pltpu.make_async_copy(
                    tspc.at[sidx.at[0]], vals.at[slot], sem).wait()

            def scatter(i, slot):
                pltpu.sync_copy(vals.at[slot], accc.at[didx.at[i]], add=True)

            issue_gather(0, 0, sem0)
            issue_gather(1, 1, sem1)

            @pl.loop(0, (NCH - 2) // 2)
            def _(j):
                for b in range(2):
                    i = j * 2 + b
                    sem = sem0 if b == 0 else sem1
                    wait_gather(b, sem)
                    scatter(i, b)
                    issue_gather(i + 2, b, sem)

            wait_gather(0, sem0)
            scatter(NCH - 2, 0)
            wait_gather(1, sem1)
            scatter(NCH - 1, 1)

        run_class(tsp0, acc0)
        run_class(tsp1, acc1)

        plsc.subcore_barrier()
        pltpu.sync_copy(acc0.at[sl],
                        out_hbm.at[pl.ds(c * NV * DEG_PAD + s * DPS, DPS)])
        pltpu.sync_copy(acc1.at[sl],
                        out_hbm.at[pl.ds(c * NV * DEG_PAD + DEG_PAD + s * DPS,
                                         DPS)])

    return aggn


def _dinv_block(degp_ref):
    deg = degp_ref[0] + degp_ref[1] + 1.0
    return lax.rsqrt(deg)


def _mm_scale_body(x_ref, w_ref, degp_ref, o_ref):
    h = jnp.dot(x_ref[...], w_ref[...], preferred_element_type=jnp.float32)
    o_ref[...] = h * _dinv_block(degp_ref)


def _epilogue1_body(aggp_ref, h1p_ref, degp_ref, b1_ref, w2_ref, o_ref):
    dinv = _dinv_block(degp_ref)
    s = aggp_ref[0] + aggp_ref[1] + h1p_ref[...]
    h = jnp.maximum(s * dinv + b1_ref[...], 0.0) * dinv
    o_ref[...] = jnp.dot(h, w2_ref[...], preferred_element_type=jnp.float32)


def _epilogue2_body(aggp_ref, tc_ref, degr_ref, b2_ref, o_ref):
    deg = degr_ref[0] + degr_ref[1] + 1.0
    dinv = lax.rsqrt(deg)
    s = aggp_ref[0] + aggp_ref[1] + tc_ref[...]
    o_ref[...] = s * dinv + b2_ref[...]


_GRID = N // RB
CB = 1024
_GRID2 = DEG_PAD // CB
_spec_rows = pl.BlockSpec((RB, D), lambda i: (i, 0))
_spec_pair = pl.BlockSpec((NC, RB, D), lambda i: (0, i, 0))
_spec_cpair = pl.BlockSpec((NC, NV, CB), lambda i: (0, 0, i))
_spec_deg = pl.BlockSpec((NC, RB, 1), lambda i: (0, i, 0))
_spec_degr = pl.BlockSpec((NC, CB), lambda i: (0, i))
_spec_w = pl.BlockSpec((D, D), lambda i: (0, 0))
_spec_wn = pl.BlockSpec((D, NV), lambda i: (0, 0))
_spec_b = pl.BlockSpec((1, D), lambda i: (0, 0))
_spec_bc = pl.BlockSpec((NV, 1), lambda i: (0, 0))
_spec_t = pl.BlockSpec((RB, NV), lambda i: (i, 0))
_spec_tcol = pl.BlockSpec((NV, CB), lambda i: (0, i))
_out_rows = jax.ShapeDtypeStruct((N, D), jnp.float32)
_out_t = jax.ShapeDtypeStruct((N, NV), jnp.float32)
_out_cols = jax.ShapeDtypeStruct((NV, DEG_PAD), jnp.float32)


def kernel(x, edge_index, W1, b1, W2, b2):
    ei = edge_index.reshape(2, NW, NCH, CH)
    zrow = jnp.zeros((DPS,), jnp.float32)
    ones = jnp.ones((CH,), jnp.float32)
    zblk = jnp.zeros((ZR, D), jnp.float32)
    b1r = b1.reshape(1, D)
    b2c = b2.reshape(NV, 1)

    degp = _deg_kernel()(ei, zrow, ones)
    degp3 = degp.reshape(NC, DEG_PAD, 1)

    h1p = pl.pallas_call(
        _mm_scale_body,
        grid=(_GRID,),
        in_specs=[_spec_rows, _spec_w, _spec_deg],
        out_specs=_spec_rows,
        out_shape=_out_rows,
    )(x, W1, degp3)

    aggp1 = _agg_kernel(D)(h1p, ei, zblk)

    t = pl.pallas_call(
        _epilogue1_body,
        grid=(_GRID,),
        in_specs=[_spec_pair, _spec_rows, _spec_deg, _spec_b, _spec_wn],
        out_specs=_spec_t,
        out_shape=_out_t,
    )(aggp1, h1p, degp3, b1r, W2)

    pad = DEG_PAD - N
    tcol = jnp.concatenate([jnp.pad(t[:, 0], (0, pad)),
                            jnp.pad(t[:, 1], (0, pad))])
    aggp2 = _aggn_kernel()(tcol, ei, zrow)
    aggp2 = aggp2.reshape(NC, NV, DEG_PAD)

    outc = pl.pallas_call(
        _epilogue2_body,
        grid=(_GRID2,),
        in_specs=[_spec_cpair, _spec_tcol, _spec_degr, _spec_bc],
        out_specs=_spec_tcol,
        out_shape=_out_cols,
    )(aggp2, tcol.reshape(NV, DEG_PAD), degp.reshape(NC, DEG_PAD), b2c)

    return outc[:, :N].T
